# baseline (speedup 1.0000x reference)
import functools

import jax
import jax.numpy as jnp
from jax import lax
from jax.experimental import pallas as pl
from jax.experimental.pallas import tpu as pltpu
from jax.experimental.pallas import tpu_sc as plsc

_NC = 2
_NS = 16
_NW = _NC * _NS
_L = 16

_CH = 128
_NBUF = 4
_LG = 3


def _bcast(vec, lane):
    idx = jnp.full((_L,), lane, jnp.int32)
    return lax.gather(
        vec, idx[:, None],
        lax.GatherDimensionNumbers(
            offset_dims=(), collapsed_slice_dims=(0,), start_index_map=(0,)),
        (1,), mode=lax.GatherScatterMode.PROMISE_IN_BOUNDS)


def _transpose_sc(vol, *, d, hv, wv, c):
    dhw = d * hv * wv
    zpt = d // _NW
    yb = 8
    nblk_per_z = hv // yb
    nblk = zpt * nblk_per_z
    bp = yb * wv
    mesh = plsc.VectorSubcoreMesh(
        core_axis_name="c", subcore_axis_name="s", num_cores=_NC,
        num_subcores=_NS)

    @functools.partial(
        pl.kernel,
        out_type=jax.ShapeDtypeStruct((dhw, c), jnp.float32),
        mesh=mesh,
        compiler_params=pltpu.CompilerParams(
            needs_layout_passes=False, use_tc_tiling_on_sc=False),
        scratch_types=[
            pltpu.VMEM((2, c, yb, wv), jnp.float32),
            pltpu.VMEM((2, bp, c), jnp.float32),
            [pltpu.SemaphoreType.DMA] * 2,
            [pltpu.SemaphoreType.DMA] * 2,
        ],
    )
    def k(vol_hbm, table_hbm, tbuf, obuf, sem_in, sem_out):
        wid = lax.axis_index("s") * _NC + lax.axis_index("c")
        z0 = wid * zpt
        lanes = lax.iota(jnp.int32, _L)

        def fire_in(blk, s):
            z = z0 + blk // nblk_per_z
            y0 = (blk % nblk_per_z) * yb
            for cc in range(c):
                pltpu.async_copy(
                    vol_hbm.at[0, cc, z, pl.ds(y0, yb)], tbuf.at[s, cc],
                    sem_in[s])

        def wait_in(blk, s):
            z = z0 + blk // nblk_per_z
            y0 = (blk % nblk_per_z) * yb
            for cc in range(c):
                pltpu.make_async_copy(
                    vol_hbm.at[0, cc, z, pl.ds(y0, yb)], tbuf.at[s, cc],
                    sem_in[s]).wait()

        def transpose(s):
            sv = jnp.full((_L,), s, jnp.int32)

            def yrow(y, _):
                yv = jnp.full((_L,), y, jnp.int32)

                def xgrp(j, _):
                    pvec = y * wv + j * _L + lanes
                    xv = j * _L + lanes
                    for cc in range(c):
                        ccv = jnp.full((_L,), cc, jnp.int32)
                        vals = plsc.load_gather(tbuf, [sv, ccv, yv, xv])
                        plsc.store_scatter(obuf, [sv, pvec, ccv], vals)
                    return 0

                lax.fori_loop(0, wv // _L, xgrp, 0)
                return 0

            lax.fori_loop(0, yb, yrow, 0)

        def fire_out(blk, s):
            z = z0 + blk // nblk_per_z
            y0 = (blk % nblk_per_z) * yb
            base = (z * hv + y0) * wv
            pltpu.async_copy(
                obuf.at[s], table_hbm.at[pl.ds(base, bp)], sem_out[s])

        def wait_out(blk, s):
            z = z0 + blk // nblk_per_z
            y0 = (blk % nblk_per_z) * yb
            base = (z * hv + y0) * wv
            pltpu.make_async_copy(
                obuf.at[s], table_hbm.at[pl.ds(base, bp)], sem_out[s]).wait()

        fire_in(0, 0)
        fire_in(1, 1)

        def step(i, _):
            for b in range(2):
                blk = 2 * i + b
                wait_in(blk, b)

                @pl.when(blk >= 2)
                def _():
                    wait_out(blk - 2, b)

                transpose(b)
                fire_out(blk, b)

                @pl.when(blk + 2 < nblk)
                def _():
                    fire_in(blk + 2, b)
            return 0

        lax.fori_loop(0, nblk // 2, step, 0)
        wait_out(nblk - 2, 0)
        wait_out(nblk - 1, 1)

    return k(vol)


def _trilinear_sc(coords, table, *, n_pts, d, hv, wv, c):
    np_per_w = n_pts // _NW
    nch = np_per_w // _CH
    assert nch % _NBUF == 0
    mesh = plsc.VectorSubcoreMesh(
        core_axis_name="c", subcore_axis_name="s", num_cores=_NC,
        num_subcores=_NS)

    @functools.partial(
        pl.kernel,
        out_type=jax.ShapeDtypeStruct((n_pts, c), jnp.float32),
        mesh=mesh,
        compiler_params=pltpu.CompilerParams(
            needs_layout_passes=False, use_tc_tiling_on_sc=False),
        scratch_types=[
            pltpu.VMEM((_NBUF, 3 * _CH), jnp.float32),
            pltpu.VMEM((_NBUF, 8, _CH), jnp.int32),
            pltpu.VMEM((_NBUF, 8, _CH), jnp.float32),
            pltpu.VMEM((_NBUF, 8 * _CH, 16), jnp.float32),
            pltpu.VMEM((_NBUF, _CH, 16), jnp.float32),
            [pltpu.SemaphoreType.DMA] * _NBUF,
            [pltpu.SemaphoreType.DMA] * _NBUF,
            [pltpu.SemaphoreType.DMA] * _NBUF,
        ],
    )
    def k(coords_hbm, table_hbm, out_hbm,
          cbuf, idxbuf, wbuf, rbuf, obuf, sem_c, sem_g, sem_o):
        wid = lax.axis_index("s") * _NC + lax.axis_index("c")
        base0 = wid * np_per_w
        lanes = lax.iota(jnp.int32, _L)

        h0 = wid * nch

        def fire_coords(g, s):
            pltpu.async_copy(coords_hbm.at[h0 + g], cbuf.at[s], sem_c[s])

        def wait_coords(g, s):
            pltpu.make_async_copy(
                coords_hbm.at[h0 + g], cbuf.at[s], sem_c[s]).wait()

        def compute_idx(s):
            sv = jnp.full((_L,), s, jnp.int32)
            for i in range(_CH // _L):
                sl = pl.ds(i * _L, _L)
                pv3 = (i * _L + lanes) * 3
                x = plsc.load_gather(cbuf, [sv, pv3])
                y = plsc.load_gather(cbuf, [sv, pv3 + 1])
                z = plsc.load_gather(cbuf, [sv, pv3 + 2])
                ix = (x * 2.0 - 1.0 + 1.0) * 0.5 * float(wv - 1)
                iy = (y * 2.0 - 1.0 + 1.0) * 0.5 * float(hv - 1)
                iz = (z * 2.0 - 1.0 + 1.0) * 0.5 * float(d - 1)
                x0 = ix.astype(jnp.int32)
                y0 = iy.astype(jnp.int32)
                z0 = iz.astype(jnp.int32)
                wx1 = ix - x0.astype(jnp.float32)
                wy1 = iy - y0.astype(jnp.float32)
                wz1 = iz - z0.astype(jnp.float32)
                wx0 = 1.0 - wx1
                wy0 = 1.0 - wy1
                wz0 = 1.0 - wz1
                xc0 = jnp.minimum(jnp.maximum(x0, 0), wv - 1)
                yc0 = jnp.minimum(jnp.maximum(y0, 0), hv - 1)
                zc0 = jnp.minimum(jnp.maximum(z0, 0), d - 1)
                xc1 = jnp.minimum(x0 + 1, wv - 1)
                yc1 = jnp.minimum(y0 + 1, hv - 1)
                zc1 = jnp.minimum(z0 + 1, d - 1)
                ty0 = yc0 * wv
                ty1 = yc1 * wv
                tz0 = zc0 * (hv * wv)
                tz1 = zc1 * (hv * wv)
                idxbuf[s, 0, sl] = tz0 + ty0 + xc0
                idxbuf[s, 1, sl] = tz0 + ty0 + xc1
                idxbuf[s, 2, sl] = tz0 + ty1 + xc0
                idxbuf[s, 3, sl] = tz0 + ty1 + xc1
                idxbuf[s, 4, sl] = tz1 + ty0 + xc0
                idxbuf[s, 5, sl] = tz1 + ty0 + xc1
                idxbuf[s, 6, sl] = tz1 + ty1 + xc0
                idxbuf[s, 7, sl] = tz1 + ty1 + xc1
                wzy00 = wz0 * wy0
                wzy01 = wz0 * wy1
                wzy10 = wz1 * wy0
                wzy11 = wz1 * wy1
                wbuf[s, 0, sl] = wzy00 * wx0
                wbuf[s, 1, sl] = wzy00 * wx1
                wbuf[s, 2, sl] = wzy01 * wx0
                wbuf[s, 3, sl] = wzy01 * wx1
                wbuf[s, 4, sl] = wzy10 * wx0
                wbuf[s, 5, sl] = wzy10 * wx1
                wbuf[s, 6, sl] = wzy11 * wx0
                wbuf[s, 7, sl] = wzy11 * wx1

        def fire_gathers(s):
            for kk in range(8):
                pltpu.async_copy(
                    table_hbm.at[idxbuf.at[s, kk]],
                    rbuf.at[s, pl.ds(kk * _CH, _CH)], sem_g[s])

        def wait_gathers(s):
            for kk in range(8):
                pltpu.make_async_copy(
                    table_hbm.at[idxbuf.at[s, kk]],
                    rbuf.at[s, pl.ds(kk * _CH, _CH)], sem_g[s]).wait()

        def combine(s):
            def pgroup(i, _):
                sl = pl.ds(i * _L, _L)
                wv8 = [wbuf[s, kk, sl] for kk in range(8)]
                for p0 in range(_L):
                    row = i * _L + p0
                    t = [_bcast(wv8[kk], p0) * rbuf[s, kk * _CH + row, :]
                         for kk in range(8)]
                    acc = (((t[0] + t[1]) + (t[2] + t[3]))
                           + ((t[4] + t[5]) + (t[6] + t[7])))
                    obuf[s, row, :] = acc
                return 0

            lax.fori_loop(0, _CH // _L, pgroup, 0)

        def fire_out(g, s):
            base = base0 + g * _CH
            pltpu.async_copy(
                obuf.at[s], out_hbm.at[pl.ds(base, _CH)], sem_o[s])

        def wait_out(g, s):
            base = base0 + g * _CH
            pltpu.make_async_copy(
                obuf.at[s], out_hbm.at[pl.ds(base, _CH)], sem_o[s]).wait()

        for c0 in range(_NBUF):
            fire_coords(c0, c0)
        for f in range(_LG):
            wait_coords(f, f)
            compute_idx(f)
            fire_gathers(f)
            fire_coords(f + _NBUF, f)

        def step(i, _):
            for b in range(_NBUF):
                g = i * _NBUF + b
                f = g + _LG
                sf = (b + _LG) % _NBUF

                @pl.when(f < nch)
                def _():
                    wait_coords(f, sf)
                    compute_idx(sf)
                    fire_gathers(sf)

                    @pl.when(f + _NBUF < nch)
                    def _():
                        fire_coords(f + _NBUF, sf)

                wait_gathers(b)

                @pl.when(g >= _NBUF)
                def _():
                    wait_out(g - _NBUF, b)

                combine(b)
                fire_out(g, b)
            return 0

        lax.fori_loop(0, nch // _NBUF, step, 0)
        for b in range(_NBUF):
            wait_out(nch - _NBUF + b, b)

    return k(coords, table)


def kernel(ray_coordinate_ref, feat_volume):
    h = ray_coordinate_ref.shape[-3]
    w = ray_coordinate_ref.shape[-2]
    b, c, d, hv, wv = feat_volume.shape
    n = h * w
    coords = ray_coordinate_ref.reshape(h, w * 3)
    table = _transpose_sc(feat_volume, d=d, hv=hv, wv=wv, c=c)
    out = _trilinear_sc(coords, table, n_pts=n, d=d, hv=hv, wv=wv, c=c)
    return out.reshape(h, w, c)


# speedup vs baseline: 1.0319x; 1.0319x over previous
import functools

import jax
import jax.numpy as jnp
from jax import lax
from jax.experimental import pallas as pl
from jax.experimental.pallas import tpu as pltpu
from jax.experimental.pallas import tpu_sc as plsc

_NC = 2
_NS = 16
_NW = _NC * _NS
_L = 16

_CH = 128
_NBUF = 4
_LG = 3


def _bcast(vec, lane):
    idx = jnp.full((_L,), lane, jnp.int32)
    return lax.gather(
        vec, idx[:, None],
        lax.GatherDimensionNumbers(
            offset_dims=(), collapsed_slice_dims=(0,), start_index_map=(0,)),
        (1,), mode=lax.GatherScatterMode.PROMISE_IN_BOUNDS)


def _transpose_sc(vol, *, d, hv, wv, c):
    dhw = d * hv * wv
    zpt = d // _NW
    yb = 8
    nblk_per_z = hv // yb
    nblk = zpt * nblk_per_z
    bp = yb * wv
    mesh = plsc.VectorSubcoreMesh(
        core_axis_name="c", subcore_axis_name="s", num_cores=_NC,
        num_subcores=_NS)

    @functools.partial(
        pl.kernel,
        out_type=jax.ShapeDtypeStruct((dhw, c), jnp.float32),
        mesh=mesh,
        compiler_params=pltpu.CompilerParams(
            needs_layout_passes=False, use_tc_tiling_on_sc=False),
        scratch_types=[
            pltpu.VMEM((2, c, yb, wv), jnp.float32),
            pltpu.VMEM((2, bp, c), jnp.float32),
            [pltpu.SemaphoreType.DMA] * 2,
            [pltpu.SemaphoreType.DMA] * 2,
        ],
    )
    def k(vol_hbm, table_hbm, tbuf, obuf, sem_in, sem_out):
        wid = lax.axis_index("s") * _NC + lax.axis_index("c")
        z0 = wid * zpt
        lanes = lax.iota(jnp.int32, _L)

        def fire_in(blk, s):
            z = z0 + blk // nblk_per_z
            y0 = (blk % nblk_per_z) * yb
            for cc in range(c):
                pltpu.async_copy(
                    vol_hbm.at[0, cc, z, pl.ds(y0, yb)], tbuf.at[s, cc],
                    sem_in[s])

        def wait_in(blk, s):
            z = z0 + blk // nblk_per_z
            y0 = (blk % nblk_per_z) * yb
            for cc in range(c):
                pltpu.make_async_copy(
                    vol_hbm.at[0, cc, z, pl.ds(y0, yb)], tbuf.at[s, cc],
                    sem_in[s]).wait()

        def transpose(s):
            sv = jnp.full((_L,), s, jnp.int32)

            def yrow(y, _):
                yv = jnp.full((_L,), y, jnp.int32)

                def xgrp(j, _):
                    pvec = y * wv + j * _L + lanes
                    xv = j * _L + lanes
                    for cc in range(c):
                        ccv = jnp.full((_L,), cc, jnp.int32)
                        vals = plsc.load_gather(tbuf, [sv, ccv, yv, xv])
                        plsc.store_scatter(obuf, [sv, pvec, ccv], vals)
                    return 0

                lax.fori_loop(0, wv // _L, xgrp, 0)
                return 0

            lax.fori_loop(0, yb, yrow, 0)

        def fire_out(blk, s):
            z = z0 + blk // nblk_per_z
            y0 = (blk % nblk_per_z) * yb
            base = (z * hv + y0) * wv
            pltpu.async_copy(
                obuf.at[s], table_hbm.at[pl.ds(base, bp)], sem_out[s])

        def wait_out(blk, s):
            z = z0 + blk // nblk_per_z
            y0 = (blk % nblk_per_z) * yb
            base = (z * hv + y0) * wv
            pltpu.make_async_copy(
                obuf.at[s], table_hbm.at[pl.ds(base, bp)], sem_out[s]).wait()

        fire_in(0, 0)
        fire_in(1, 1)

        def step(i, _):
            for b in range(2):
                blk = 2 * i + b
                wait_in(blk, b)

                @pl.when(blk >= 2)
                def _():
                    wait_out(blk - 2, b)

                transpose(b)
                fire_out(blk, b)

                @pl.when(blk + 2 < nblk)
                def _():
                    fire_in(blk + 2, b)
            return 0

        lax.fori_loop(0, nblk // 2, step, 0)
        wait_out(nblk - 2, 0)
        wait_out(nblk - 1, 1)

    return k(vol)


def _trilinear_sc(coords, table, *, n_pts, d, hv, wv, c):
    np_per_w = n_pts // _NW
    nch = np_per_w // _CH
    assert nch % _NBUF == 0
    mesh = plsc.VectorSubcoreMesh(
        core_axis_name="c", subcore_axis_name="s", num_cores=_NC,
        num_subcores=_NS)

    @functools.partial(
        pl.kernel,
        out_type=jax.ShapeDtypeStruct((n_pts, c), jnp.float32),
        mesh=mesh,
        compiler_params=pltpu.CompilerParams(
            needs_layout_passes=False, use_tc_tiling_on_sc=False),
        scratch_types=[
            pltpu.VMEM((_NBUF, 3, _CH), jnp.float32),
            pltpu.VMEM((_NBUF, 8, _CH), jnp.int32),
            pltpu.VMEM((_NBUF, 8, _CH), jnp.float32),
            pltpu.VMEM((_NBUF, 8 * _CH, 16), jnp.float32),
            pltpu.VMEM((_NBUF, _CH, 16), jnp.float32),
            [pltpu.SemaphoreType.DMA] * _NBUF,
            [pltpu.SemaphoreType.DMA] * _NBUF,
            [pltpu.SemaphoreType.DMA] * _NBUF,
        ],
    )
    def k(coords_hbm, table_hbm, out_hbm,
          cbuf, idxbuf, wbuf, rbuf, obuf, sem_c, sem_g, sem_o):
        wid = lax.axis_index("s") * _NC + lax.axis_index("c")
        base0 = wid * np_per_w
        lanes = lax.iota(jnp.int32, _L)

        def fire_coords(g, s):
            base = base0 + g * _CH
            pltpu.async_copy(
                coords_hbm.at[:, pl.ds(base, _CH)], cbuf.at[s], sem_c[s])

        def wait_coords(g, s):
            base = base0 + g * _CH
            pltpu.make_async_copy(
                coords_hbm.at[:, pl.ds(base, _CH)], cbuf.at[s],
                sem_c[s]).wait()

        def compute_idx(s):
            for i in range(_CH // _L):
                sl = pl.ds(i * _L, _L)
                x = cbuf[s, 0, sl]
                y = cbuf[s, 1, sl]
                z = cbuf[s, 2, sl]
                ix = (x * 2.0 - 1.0 + 1.0) * 0.5 * float(wv - 1)
                iy = (y * 2.0 - 1.0 + 1.0) * 0.5 * float(hv - 1)
                iz = (z * 2.0 - 1.0 + 1.0) * 0.5 * float(d - 1)
                x0 = ix.astype(jnp.int32)
                y0 = iy.astype(jnp.int32)
                z0 = iz.astype(jnp.int32)
                wx1 = ix - x0.astype(jnp.float32)
                wy1 = iy - y0.astype(jnp.float32)
                wz1 = iz - z0.astype(jnp.float32)
                wx0 = 1.0 - wx1
                wy0 = 1.0 - wy1
                wz0 = 1.0 - wz1
                xc0 = jnp.minimum(jnp.maximum(x0, 0), wv - 1)
                yc0 = jnp.minimum(jnp.maximum(y0, 0), hv - 1)
                zc0 = jnp.minimum(jnp.maximum(z0, 0), d - 1)
                xc1 = jnp.minimum(x0 + 1, wv - 1)
                yc1 = jnp.minimum(y0 + 1, hv - 1)
                zc1 = jnp.minimum(z0 + 1, d - 1)
                ty0 = yc0 * wv
                ty1 = yc1 * wv
                tz0 = zc0 * (hv * wv)
                tz1 = zc1 * (hv * wv)
                idxbuf[s, 0, sl] = tz0 + ty0 + xc0
                idxbuf[s, 1, sl] = tz0 + ty0 + xc1
                idxbuf[s, 2, sl] = tz0 + ty1 + xc0
                idxbuf[s, 3, sl] = tz0 + ty1 + xc1
                idxbuf[s, 4, sl] = tz1 + ty0 + xc0
                idxbuf[s, 5, sl] = tz1 + ty0 + xc1
                idxbuf[s, 6, sl] = tz1 + ty1 + xc0
                idxbuf[s, 7, sl] = tz1 + ty1 + xc1
                wzy00 = wz0 * wy0
                wzy01 = wz0 * wy1
                wzy10 = wz1 * wy0
                wzy11 = wz1 * wy1
                wbuf[s, 0, sl] = wzy00 * wx0
                wbuf[s, 1, sl] = wzy00 * wx1
                wbuf[s, 2, sl] = wzy01 * wx0
                wbuf[s, 3, sl] = wzy01 * wx1
                wbuf[s, 4, sl] = wzy10 * wx0
                wbuf[s, 5, sl] = wzy10 * wx1
                wbuf[s, 6, sl] = wzy11 * wx0
                wbuf[s, 7, sl] = wzy11 * wx1

        def fire_gathers(s):
            for kk in range(8):
                pltpu.async_copy(
                    table_hbm.at[idxbuf.at[s, kk]],
                    rbuf.at[s, pl.ds(kk * _CH, _CH)], sem_g[s])

        def wait_gathers(s):
            for kk in range(8):
                pltpu.make_async_copy(
                    table_hbm.at[idxbuf.at[s, kk]],
                    rbuf.at[s, pl.ds(kk * _CH, _CH)], sem_g[s]).wait()

        def combine(s):
            def pgroup(i, _):
                sl = pl.ds(i * _L, _L)
                wv8 = [wbuf[s, kk, sl] for kk in range(8)]
                for p0 in range(_L):
                    row = i * _L + p0
                    t = [_bcast(wv8[kk], p0) * rbuf[s, kk * _CH + row, :]
                         for kk in range(8)]
                    acc = (((t[0] + t[1]) + (t[2] + t[3]))
                           + ((t[4] + t[5]) + (t[6] + t[7])))
                    obuf[s, row, :] = acc
                return 0

            lax.fori_loop(0, _CH // _L, pgroup, 0)

        def fire_out(g, s):
            base = base0 + g * _CH
            pltpu.async_copy(
                obuf.at[s], out_hbm.at[pl.ds(base, _CH)], sem_o[s])

        def wait_out(g, s):
            base = base0 + g * _CH
            pltpu.make_async_copy(
                obuf.at[s], out_hbm.at[pl.ds(base, _CH)], sem_o[s]).wait()

        for c0 in range(_NBUF):
            fire_coords(c0, c0)
        for f in range(_LG):
            wait_coords(f, f)
            compute_idx(f)
            fire_gathers(f)
            fire_coords(f + _NBUF, f)

        def step(i, _):
            for b in range(_NBUF):
                g = i * _NBUF + b
                f = g + _LG
                sf = (b + _LG) % _NBUF

                @pl.when(f < nch)
                def _():
                    wait_coords(f, sf)
                    compute_idx(sf)
                    fire_gathers(sf)

                    @pl.when(f + _NBUF < nch)
                    def _():
                        fire_coords(f + _NBUF, sf)

                wait_gathers(b)

                @pl.when(g >= _NBUF)
                def _():
                    wait_out(g - _NBUF, b)

                combine(b)
                fire_out(g, b)
            return 0

        lax.fori_loop(0, nch // _NBUF, step, 0)
        for b in range(_NBUF):
            wait_out(nch - _NBUF + b, b)

    return k(coords, table)


def kernel(ray_coordinate_ref, feat_volume):
    h = ray_coordinate_ref.shape[-3]
    w = ray_coordinate_ref.shape[-2]
    b, c, d, hv, wv = feat_volume.shape
    n = h * w
    coords = jnp.transpose(ray_coordinate_ref.reshape(n, 3))
    table = _transpose_sc(feat_volume, d=d, hv=hv, wv=wv, c=c)
    out = _trilinear_sc(coords, table, n_pts=n, d=d, hv=hv, wv=wv, c=c)
    return out.reshape(h, w, c)


# speedup vs baseline: 1.0753x; 1.0421x over previous
import functools

import jax
import jax.numpy as jnp
from jax import lax
from jax.experimental import pallas as pl
from jax.experimental.pallas import tpu as pltpu
from jax.experimental.pallas import tpu_sc as plsc

_NC = 2
_NS = 16
_NW = _NC * _NS
_L = 16

_CH = 128
_NBUF = 4
_LG = 3


def _bcast(vec, lane):
    idx = jnp.full((_L,), lane, jnp.int32)
    return lax.gather(
        vec, idx[:, None],
        lax.GatherDimensionNumbers(
            offset_dims=(), collapsed_slice_dims=(0,), start_index_map=(0,)),
        (1,), mode=lax.GatherScatterMode.PROMISE_IN_BOUNDS)


def _transpose_sc(vol, *, d, hv, wv, c):
    dhw = d * hv * wv
    zpt = d // _NW
    yb = 8
    nblk_per_z = hv // yb
    nblk = zpt * nblk_per_z
    bp = yb * wv
    mesh = plsc.VectorSubcoreMesh(
        core_axis_name="c", subcore_axis_name="s", num_cores=_NC,
        num_subcores=_NS)

    @functools.partial(
        pl.kernel,
        out_type=jax.ShapeDtypeStruct((dhw, c), jnp.float32),
        mesh=mesh,
        compiler_params=pltpu.CompilerParams(
            needs_layout_passes=False, use_tc_tiling_on_sc=False),
        scratch_types=[
            pltpu.VMEM((2, c, yb, wv), jnp.float32),
            pltpu.VMEM((2, bp, c), jnp.float32),
            [pltpu.SemaphoreType.DMA] * 2,
            [pltpu.SemaphoreType.DMA] * 2,
        ],
    )
    def k(vol_hbm, table_hbm, tbuf, obuf, sem_in, sem_out):
        wid = lax.axis_index("s") * _NC + lax.axis_index("c")
        z0 = wid * zpt
        lanes = lax.iota(jnp.int32, _L)

        def fire_in(blk, s):
            z = z0 + blk // nblk_per_z
            y0 = (blk % nblk_per_z) * yb
            for cc in range(c):
                pltpu.async_copy(
                    vol_hbm.at[0, cc, z, pl.ds(y0, yb)], tbuf.at[s, cc],
                    sem_in[s])

        def wait_in(blk, s):
            z = z0 + blk // nblk_per_z
            y0 = (blk % nblk_per_z) * yb
            for cc in range(c):
                pltpu.make_async_copy(
                    vol_hbm.at[0, cc, z, pl.ds(y0, yb)], tbuf.at[s, cc],
                    sem_in[s]).wait()

        def transpose(s):
            sv = jnp.full((_L,), s, jnp.int32)

            def yrow(y, _):
                yv = jnp.full((_L,), y, jnp.int32)

                def xgrp(j):
                    pvec = y * wv + j * _L + lanes
                    xv = j * _L + lanes
                    for cc in range(c):
                        ccv = jnp.full((_L,), cc, jnp.int32)
                        vals = plsc.load_gather(tbuf, [sv, ccv, yv, xv])
                        plsc.store_scatter(obuf, [sv, pvec, ccv], vals)

                plsc.parallel_loop(0, wv // _L, 1, unroll=2)(xgrp)
                return 0

            lax.fori_loop(0, yb, yrow, 0)

        def fire_out(blk, s):
            z = z0 + blk // nblk_per_z
            y0 = (blk % nblk_per_z) * yb
            base = (z * hv + y0) * wv
            pltpu.async_copy(
                obuf.at[s], table_hbm.at[pl.ds(base, bp)], sem_out[s])

        def wait_out(blk, s):
            z = z0 + blk // nblk_per_z
            y0 = (blk % nblk_per_z) * yb
            base = (z * hv + y0) * wv
            pltpu.make_async_copy(
                obuf.at[s], table_hbm.at[pl.ds(base, bp)], sem_out[s]).wait()

        fire_in(0, 0)
        fire_in(1, 1)

        def step(i, _):
            for b in range(2):
                blk = 2 * i + b
                wait_in(blk, b)

                @pl.when(blk >= 2)
                def _():
                    wait_out(blk - 2, b)

                transpose(b)
                fire_out(blk, b)

                @pl.when(blk + 2 < nblk)
                def _():
                    fire_in(blk + 2, b)
            return 0

        lax.fori_loop(0, nblk // 2, step, 0)
        wait_out(nblk - 2, 0)
        wait_out(nblk - 1, 1)

    return k(vol)


def _trilinear_sc(coords, table, *, n_pts, d, hv, wv, c):
    np_per_w = n_pts // _NW
    nch = np_per_w // _CH
    assert nch % _NBUF == 0
    mesh = plsc.VectorSubcoreMesh(
        core_axis_name="c", subcore_axis_name="s", num_cores=_NC,
        num_subcores=_NS)

    @functools.partial(
        pl.kernel,
        out_type=jax.ShapeDtypeStruct((n_pts, c), jnp.float32),
        mesh=mesh,
        compiler_params=pltpu.CompilerParams(
            needs_layout_passes=False, use_tc_tiling_on_sc=False),
        scratch_types=[
            pltpu.VMEM((_NBUF, 3, _CH), jnp.float32),
            pltpu.VMEM((_NBUF, 8, _CH), jnp.int32),
            pltpu.VMEM((_NBUF, 8, _CH), jnp.float32),
            pltpu.VMEM((_NBUF, 8 * _CH, 16), jnp.float32),
            pltpu.VMEM((_NBUF, _CH, 16), jnp.float32),
            [pltpu.SemaphoreType.DMA] * _NBUF,
            [pltpu.SemaphoreType.DMA] * _NBUF,
            [pltpu.SemaphoreType.DMA] * _NBUF,
        ],
    )
    def k(coords_hbm, table_hbm, out_hbm,
          cbuf, idxbuf, wbuf, rbuf, obuf, sem_c, sem_g, sem_o):
        wid = lax.axis_index("s") * _NC + lax.axis_index("c")
        base0 = wid * np_per_w
        lanes = lax.iota(jnp.int32, _L)

        def fire_coords(g, s):
            base = base0 + g * _CH
            pltpu.async_copy(
                coords_hbm.at[:, pl.ds(base, _CH)], cbuf.at[s], sem_c[s])

        def wait_coords(g, s):
            base = base0 + g * _CH
            pltpu.make_async_copy(
                coords_hbm.at[:, pl.ds(base, _CH)], cbuf.at[s],
                sem_c[s]).wait()

        def compute_idx(s):
            for i in range(_CH // _L):
                sl = pl.ds(i * _L, _L)
                x = cbuf[s, 0, sl]
                y = cbuf[s, 1, sl]
                z = cbuf[s, 2, sl]
                ix = (x * 2.0 - 1.0 + 1.0) * 0.5 * float(wv - 1)
                iy = (y * 2.0 - 1.0 + 1.0) * 0.5 * float(hv - 1)
                iz = (z * 2.0 - 1.0 + 1.0) * 0.5 * float(d - 1)
                x0 = ix.astype(jnp.int32)
                y0 = iy.astype(jnp.int32)
                z0 = iz.astype(jnp.int32)
                wx1 = ix - x0.astype(jnp.float32)
                wy1 = iy - y0.astype(jnp.float32)
                wz1 = iz - z0.astype(jnp.float32)
                wx0 = 1.0 - wx1
                wy0 = 1.0 - wy1
                wz0 = 1.0 - wz1
                xc0 = jnp.minimum(jnp.maximum(x0, 0), wv - 1)
                yc0 = jnp.minimum(jnp.maximum(y0, 0), hv - 1)
                zc0 = jnp.minimum(jnp.maximum(z0, 0), d - 1)
                xc1 = jnp.minimum(x0 + 1, wv - 1)
                yc1 = jnp.minimum(y0 + 1, hv - 1)
                zc1 = jnp.minimum(z0 + 1, d - 1)
                ty0 = yc0 * wv
                ty1 = yc1 * wv
                tz0 = zc0 * (hv * wv)
                tz1 = zc1 * (hv * wv)
                idxbuf[s, 0, sl] = tz0 + ty0 + xc0
                idxbuf[s, 1, sl] = tz0 + ty0 + xc1
                idxbuf[s, 2, sl] = tz0 + ty1 + xc0
                idxbuf[s, 3, sl] = tz0 + ty1 + xc1
                idxbuf[s, 4, sl] = tz1 + ty0 + xc0
                idxbuf[s, 5, sl] = tz1 + ty0 + xc1
                idxbuf[s, 6, sl] = tz1 + ty1 + xc0
                idxbuf[s, 7, sl] = tz1 + ty1 + xc1
                wzy00 = wz0 * wy0
                wzy01 = wz0 * wy1
                wzy10 = wz1 * wy0
                wzy11 = wz1 * wy1
                wbuf[s, 0, sl] = wzy00 * wx0
                wbuf[s, 1, sl] = wzy00 * wx1
                wbuf[s, 2, sl] = wzy01 * wx0
                wbuf[s, 3, sl] = wzy01 * wx1
                wbuf[s, 4, sl] = wzy10 * wx0
                wbuf[s, 5, sl] = wzy10 * wx1
                wbuf[s, 6, sl] = wzy11 * wx0
                wbuf[s, 7, sl] = wzy11 * wx1

        def fire_gathers(s):
            for kk in range(8):
                pltpu.async_copy(
                    table_hbm.at[idxbuf.at[s, kk]],
                    rbuf.at[s, pl.ds(kk * _CH, _CH)], sem_g[s])

        def wait_gathers(s):
            for kk in range(8):
                pltpu.make_async_copy(
                    table_hbm.at[idxbuf.at[s, kk]],
                    rbuf.at[s, pl.ds(kk * _CH, _CH)], sem_g[s]).wait()

        def combine(s):
            def pgroup(i):
                sl = pl.ds(i * _L, _L)
                wv8 = [wbuf[s, kk, sl] for kk in range(8)]
                for p0 in range(_L):
                    row = i * _L + p0
                    t = [_bcast(wv8[kk], p0) * rbuf[s, kk * _CH + row, :]
                         for kk in range(8)]
                    acc = (((t[0] + t[1]) + (t[2] + t[3]))
                           + ((t[4] + t[5]) + (t[6] + t[7])))
                    obuf[s, row, :] = acc

            plsc.parallel_loop(0, _CH // _L, 1, unroll=2)(pgroup)

        def fire_out(g, s):
            base = base0 + g * _CH
            pltpu.async_copy(
                obuf.at[s], out_hbm.at[pl.ds(base, _CH)], sem_o[s])

        def wait_out(g, s):
            base = base0 + g * _CH
            pltpu.make_async_copy(
                obuf.at[s], out_hbm.at[pl.ds(base, _CH)], sem_o[s]).wait()

        for c0 in range(_NBUF):
            fire_coords(c0, c0)
        for f in range(_LG):
            wait_coords(f, f)
            compute_idx(f)
            fire_gathers(f)
            fire_coords(f + _NBUF, f)

        def step(i, _):
            for b in range(_NBUF):
                g = i * _NBUF + b
                f = g + _LG
                sf = (b + _LG) % _NBUF

                @pl.when(f < nch)
                def _():
                    wait_coords(f, sf)
                    compute_idx(sf)
                    fire_gathers(sf)

                    @pl.when(f + _NBUF < nch)
                    def _():
                        fire_coords(f + _NBUF, sf)

                wait_gathers(b)

                @pl.when(g >= _NBUF)
                def _():
                    wait_out(g - _NBUF, b)

                combine(b)
                fire_out(g, b)
            return 0

        lax.fori_loop(0, nch // _NBUF, step, 0)
        for b in range(_NBUF):
            wait_out(nch - _NBUF + b, b)

    return k(coords, table)


def kernel(ray_coordinate_ref, feat_volume):
    h = ray_coordinate_ref.shape[-3]
    w = ray_coordinate_ref.shape[-2]
    b, c, d, hv, wv = feat_volume.shape
    n = h * w
    coords = jnp.transpose(ray_coordinate_ref.reshape(n, 3))
    table = _transpose_sc(feat_volume, d=d, hv=hv, wv=wv, c=c)
    out = _trilinear_sc(coords, table, n_pts=n, d=d, hv=hv, wv=wv, c=c)
    return out.reshape(h, w, c)
